# baseline (speedup 1.0000x reference)
import functools

import jax
import jax.numpy as jnp
from jax import lax
from jax.experimental import pallas as pl
from jax.experimental.pallas import tpu as pltpu
from jax.experimental.pallas import tpu_sc as plsc

E = 12
K = 2
D = 1024
F = 4096
T = 2048
CAP = (T * K // E) * 2
CAP_PAD = 768
R_BLK = 128
N_RBLK = CAP_PAD // R_BLK
FF_BLK = 512
N_FFBLK = F // FF_BLK
NROWS = E * CAP_PAD
OVERFLOW = NROWS
EL = 128
CHUNK = 128
N_CHUNK = T // CHUNK


def _router_body(x_ref, wr_ref, s0s_ref, s1s_ref, s0g_ref, s1g_ref,
                 g0_ref, g1_ref, cnt_ref):
    x = x_ref[...]
    logits = jnp.dot(x, wr_ref[...], preferred_element_type=jnp.float32)
    col = lax.broadcasted_iota(jnp.int32, (T, EL), 1)
    valid = col < E
    neg = jnp.float32(-1e30)
    logits = jnp.where(valid, logits, neg)
    m = jnp.max(logits, axis=1, keepdims=True)
    ex = jnp.where(valid, jnp.exp(logits - m), 0.0)
    s = jnp.sum(ex, axis=1, keepdims=True)
    p = ex / s
    m0 = jnp.max(p, axis=1, keepdims=True)
    i0 = jnp.min(jnp.where(p == m0, col, EL), axis=1, keepdims=True)
    p2 = jnp.where(col == i0, -1.0, p)
    m1 = jnp.max(p2, axis=1, keepdims=True)
    i1 = jnp.min(jnp.where(p2 == m1, col, EL), axis=1, keepdims=True)
    denom = m0 + m1
    g0 = m0 / denom
    g1 = m1 / denom
    oh0 = (col == i0).astype(jnp.float32)
    oh1 = (col == i1).astype(jnp.float32)
    ohsum = oh0 + oh1
    ri = lax.broadcasted_iota(jnp.int32, (CHUNK, CHUNK), 0)
    ci = lax.broadcasted_iota(jnp.int32, (CHUNK, CHUNK), 1)
    lstrict = (ci < ri).astype(jnp.float32)
    off = jnp.zeros((1, EL), jnp.float32)
    parts = []
    for ci in range(N_CHUNK):
        blk = ohsum[ci * CHUNK:(ci + 1) * CHUNK, :]
        parts.append(jnp.dot(lstrict, blk, preferred_element_type=jnp.float32)
                     + off)
        off = off + jnp.sum(blk, axis=0, keepdims=True)
    cex = jnp.concatenate(parts, axis=0)
    pos0 = jnp.sum(cex * oh0, axis=1, keepdims=True)
    pos1 = jnp.sum(cex * oh1, axis=1, keepdims=True)
    e0 = i0.astype(jnp.float32)
    e1 = i1.astype(jnp.float32)
    keep0 = pos0 < CAP
    keep1 = pos1 < CAP
    slot0 = e0 * CAP_PAD + pos0
    slot1 = e1 * CAP_PAD + pos1
    s0s_ref[...] = jnp.where(keep0, slot0, OVERFLOW).astype(jnp.int32)
    s1s_ref[...] = jnp.where(keep1, slot1, OVERFLOW).astype(jnp.int32)
    s0g_ref[...] = jnp.where(keep0, slot0, 0.0).astype(jnp.int32)
    s1g_ref[...] = jnp.where(keep1, slot1, 0.0).astype(jnp.int32)
    g0_ref[...] = g0 * keep0.astype(jnp.float32)
    g1_ref[...] = g1 * keep1.astype(jnp.float32)
    cnt_ref[...] = jnp.minimum(off, float(CAP)).astype(jnp.int32)


def _router(x, wr_pad):
    out_shapes = (
        jax.ShapeDtypeStruct((T, 1), jnp.int32),
        jax.ShapeDtypeStruct((T, 1), jnp.int32),
        jax.ShapeDtypeStruct((T, 1), jnp.int32),
        jax.ShapeDtypeStruct((T, 1), jnp.int32),
        jax.ShapeDtypeStruct((T, 1), jnp.float32),
        jax.ShapeDtypeStruct((T, 1), jnp.float32),
        jax.ShapeDtypeStruct((1, EL), jnp.int32),
    )
    return pl.pallas_call(_router_body, out_shape=out_shapes)(x, wr_pad)


def _sc_mesh():
    return plsc.VectorSubcoreMesh(core_axis_name="c", subcore_axis_name="s")


_NW = 32
_TPW = T // _NW


def _dispatch(x, idx0, idx1):
    @functools.partial(
        pl.kernel,
        out_type=jax.ShapeDtypeStruct((NROWS + 8, D), jnp.float32),
        mesh=_sc_mesh(),
        scratch_types=[
            pltpu.VMEM((_TPW,), jnp.int32),
            pltpu.VMEM((_TPW,), jnp.int32),
            pltpu.VMEM((_TPW, D), jnp.float32),
            pltpu.SemaphoreType.DMA,
        ],
    )
    def scatter_kernel(x_hbm, i0_hbm, i1_hbm, buf_hbm, i0_v, i1_v, rows_v,
                       sem):
        wid = lax.axis_index("s") * 2 + lax.axis_index("c")
        base = wid * _TPW
        pltpu.sync_copy(i0_hbm.at[pl.ds(base, _TPW)], i0_v)
        pltpu.sync_copy(i1_hbm.at[pl.ds(base, _TPW)], i1_v)
        pltpu.sync_copy(x_hbm.at[pl.ds(base, _TPW)], rows_v)
        pltpu.async_copy(rows_v, buf_hbm.at[i0_v], sem).wait()
        pltpu.async_copy(rows_v, buf_hbm.at[i1_v], sem).wait()

    return scatter_kernel(x, idx0, idx1)


def _ffn_body(cnt_ref, buf_ref, w1_ref, w2_ref, y_ref, xb_ref, w1b_ref,
              w2b_ref):
    e = pl.program_id(0)
    ff = pl.program_id(1)
    r = pl.program_id(2)

    @pl.when(r * R_BLK < cnt_ref[e])
    def _():
        @pl.when(ff == 0)
        def _():
            xb_ref[pl.ds(r * R_BLK, R_BLK), :] = (
                buf_ref[pl.ds(r * R_BLK, R_BLK), :].astype(jnp.bfloat16))

        @pl.when(r == 0)
        def _():
            w1b_ref[...] = w1_ref[0].astype(jnp.bfloat16)
            w2b_ref[...] = w2_ref[0].astype(jnp.bfloat16)

        xb = xb_ref[pl.ds(r * R_BLK, R_BLK), :]
        h = jnp.dot(xb, w1b_ref[...], preferred_element_type=jnp.float32)
        h = jax.nn.gelu(h).astype(jnp.bfloat16)
        y = jnp.dot(h, w2b_ref[...], preferred_element_type=jnp.float32)

        @pl.when(ff == 0)
        def _():
            y_ref[pl.ds(r * R_BLK, R_BLK), :] = y

        @pl.when(ff != 0)
        def _():
            y_ref[pl.ds(r * R_BLK, R_BLK), :] += y


def _ffn(cnt, buf, w1, w2):
    grid_spec = pltpu.PrefetchScalarGridSpec(
        num_scalar_prefetch=1,
        grid=(E, N_FFBLK, N_RBLK),
        in_specs=[
            pl.BlockSpec((CAP_PAD, D), lambda e, ff, r, cnt: (e, 0)),
            pl.BlockSpec((1, D, FF_BLK), lambda e, ff, r, cnt: (e, 0, ff)),
            pl.BlockSpec((1, FF_BLK, D), lambda e, ff, r, cnt: (e, ff, 0)),
        ],
        out_specs=pl.BlockSpec((CAP_PAD, D), lambda e, ff, r, cnt: (e, 0)),
        scratch_shapes=[
            pltpu.VMEM((CAP_PAD, D), jnp.bfloat16),
            pltpu.VMEM((D, FF_BLK), jnp.bfloat16),
            pltpu.VMEM((FF_BLK, D), jnp.bfloat16),
        ],
    )
    return pl.pallas_call(
        _ffn_body,
        grid_spec=grid_spec,
        out_shape=jax.ShapeDtypeStruct((NROWS, D), jnp.float32),
    )(cnt, buf, w1, w2)


def _gather2(y, idx0, idx1):
    out_types = (
        jax.ShapeDtypeStruct((T, D), jnp.float32),
        jax.ShapeDtypeStruct((T, D), jnp.float32),
    )

    @functools.partial(
        pl.kernel,
        out_type=out_types,
        mesh=_sc_mesh(),
        scratch_types=[
            pltpu.VMEM((_TPW,), jnp.int32),
            pltpu.VMEM((_TPW, D), jnp.float32),
            pltpu.SemaphoreType.DMA,
        ],
    )
    def gather_kernel(y_hbm, i0_hbm, i1_hbm, a_hbm, b_hbm, i_v, rows_v, sem):
        wid = lax.axis_index("s") * 2 + lax.axis_index("c")
        base = wid * _TPW
        pltpu.sync_copy(i0_hbm.at[pl.ds(base, _TPW)], i_v)
        pltpu.async_copy(y_hbm.at[i_v], rows_v, sem).wait()
        pltpu.sync_copy(rows_v, a_hbm.at[pl.ds(base, _TPW)])
        pltpu.sync_copy(i1_hbm.at[pl.ds(base, _TPW)], i_v)
        pltpu.async_copy(y_hbm.at[i_v], rows_v, sem).wait()
        pltpu.sync_copy(rows_v, b_hbm.at[pl.ds(base, _TPW)])

    return gather_kernel(y, idx0, idx1)


def _combine_body(a_ref, b_ref, g0_ref, g1_ref, o_ref):
    o_ref[...] = g0_ref[...] * a_ref[...] + g1_ref[...] * b_ref[...]


def _combine(a, b, g0, g1):
    n_blk = T // 256
    return pl.pallas_call(
        _combine_body,
        grid=(n_blk,),
        in_specs=[
            pl.BlockSpec((256, D), lambda i: (i, 0)),
            pl.BlockSpec((256, D), lambda i: (i, 0)),
            pl.BlockSpec((256, 1), lambda i: (i, 0)),
            pl.BlockSpec((256, 1), lambda i: (i, 0)),
        ],
        out_specs=pl.BlockSpec((256, D), lambda i: (i, 0)),
        out_shape=jax.ShapeDtypeStruct((T, D), jnp.float32),
    )(a, b, g0, g1)


def kernel(x, w_router, w1, w2):
    wr_pad = jnp.zeros((D, EL), jnp.float32).at[:, :E].set(w_router)
    s0s, s1s, s0g, s1g, g0, g1, cnt = _router(x, wr_pad)
    buf = _dispatch(x, s0s[:, 0], s1s[:, 0])
    y = _ffn(cnt[0], buf, w1, w2)
    a, b = _gather2(y, s0g[:, 0], s1g[:, 0])
    return _combine(a, b, g0, g1)


# speedup vs baseline: 1.5559x; 1.5559x over previous
import functools

import jax
import jax.numpy as jnp
from jax import lax
from jax.experimental import pallas as pl
from jax.experimental.pallas import tpu as pltpu
from jax.experimental.pallas import tpu_sc as plsc

E = 12
K = 2
D = 1024
F = 4096
T = 2048
CAP = (T * K // E) * 2
CAP_PAD = 768
R_BLK = 128
N_RBLK = CAP_PAD // R_BLK
FF_BLK = 512
N_FFBLK = F // FF_BLK
NROWS = E * CAP_PAD
OVERFLOW = NROWS
EL = 128
CHUNK = 128
N_CHUNK = T // CHUNK


def _router_body(x_ref, wr_ref, s0s_ref, s1s_ref, s0g_ref, s1g_ref,
                 g0_ref, g1_ref, cnt_ref):
    x = x_ref[...]
    logits = jnp.dot(x, wr_ref[...], preferred_element_type=jnp.float32)
    col = lax.broadcasted_iota(jnp.int32, (T, EL), 1)
    valid = col < E
    neg = jnp.float32(-1e30)
    logits = jnp.where(valid, logits, neg)
    m = jnp.max(logits, axis=1, keepdims=True)
    ex = jnp.where(valid, jnp.exp(logits - m), 0.0)
    s = jnp.sum(ex, axis=1, keepdims=True)
    p = ex / s
    m0 = jnp.max(p, axis=1, keepdims=True)
    i0 = jnp.min(jnp.where(p == m0, col, EL), axis=1, keepdims=True)
    p2 = jnp.where(col == i0, -1.0, p)
    m1 = jnp.max(p2, axis=1, keepdims=True)
    i1 = jnp.min(jnp.where(p2 == m1, col, EL), axis=1, keepdims=True)
    denom = m0 + m1
    g0 = m0 / denom
    g1 = m1 / denom
    oh0 = (col == i0).astype(jnp.float32)
    oh1 = (col == i1).astype(jnp.float32)
    ohsum = oh0 + oh1
    ri = lax.broadcasted_iota(jnp.int32, (CHUNK, CHUNK), 0)
    ci = lax.broadcasted_iota(jnp.int32, (CHUNK, CHUNK), 1)
    lstrict = (ci < ri).astype(jnp.float32)
    off = jnp.zeros((1, EL), jnp.float32)
    parts = []
    for ci in range(N_CHUNK):
        blk = ohsum[ci * CHUNK:(ci + 1) * CHUNK, :]
        parts.append(jnp.dot(lstrict, blk, preferred_element_type=jnp.float32)
                     + off)
        off = off + jnp.sum(blk, axis=0, keepdims=True)
    cex = jnp.concatenate(parts, axis=0)
    pos0 = jnp.sum(cex * oh0, axis=1, keepdims=True)
    pos1 = jnp.sum(cex * oh1, axis=1, keepdims=True)
    e0 = i0.astype(jnp.float32)
    e1 = i1.astype(jnp.float32)
    keep0 = pos0 < CAP
    keep1 = pos1 < CAP
    slot0 = e0 * CAP_PAD + pos0
    slot1 = e1 * CAP_PAD + pos1
    s0s_ref[...] = jnp.where(keep0, slot0, OVERFLOW).astype(jnp.int32)
    s1s_ref[...] = jnp.where(keep1, slot1, OVERFLOW).astype(jnp.int32)
    s0g_ref[...] = jnp.where(keep0, slot0, 0.0).astype(jnp.int32)
    s1g_ref[...] = jnp.where(keep1, slot1, 0.0).astype(jnp.int32)
    g0_ref[...] = g0 * keep0.astype(jnp.float32)
    g1_ref[...] = g1 * keep1.astype(jnp.float32)
    cnt_ref[...] = jnp.minimum(off, float(CAP)).astype(jnp.int32)


def _router(x, wr_pad):
    out_shapes = (
        jax.ShapeDtypeStruct((T, 1), jnp.int32),
        jax.ShapeDtypeStruct((T, 1), jnp.int32),
        jax.ShapeDtypeStruct((T, 1), jnp.int32),
        jax.ShapeDtypeStruct((T, 1), jnp.int32),
        jax.ShapeDtypeStruct((T, 1), jnp.float32),
        jax.ShapeDtypeStruct((T, 1), jnp.float32),
        jax.ShapeDtypeStruct((1, EL), jnp.int32),
    )
    return pl.pallas_call(_router_body, out_shape=out_shapes)(x, wr_pad)


def _sc_mesh():
    return plsc.VectorSubcoreMesh(core_axis_name="c", subcore_axis_name="s")


_NW = 32
_TPW = T // _NW


def _dispatch(x, idx0, idx1):
    @functools.partial(
        pl.kernel,
        out_type=jax.ShapeDtypeStruct((NROWS + 8, D), jnp.float32),
        mesh=_sc_mesh(),
        scratch_types=[
            pltpu.VMEM((_TPW,), jnp.int32),
            pltpu.VMEM((_TPW,), jnp.int32),
            pltpu.VMEM((_TPW, D), jnp.float32),
            pltpu.SemaphoreType.DMA,
        ],
    )
    def scatter_kernel(x_hbm, i0_hbm, i1_hbm, buf_hbm, i0_v, i1_v, rows_v,
                       sem):
        wid = lax.axis_index("s") * 2 + lax.axis_index("c")
        base = wid * _TPW
        pltpu.sync_copy(i0_hbm.at[pl.ds(base, _TPW)], i0_v)
        pltpu.sync_copy(i1_hbm.at[pl.ds(base, _TPW)], i1_v)
        pltpu.sync_copy(x_hbm.at[pl.ds(base, _TPW)], rows_v)
        pltpu.async_copy(rows_v, buf_hbm.at[i0_v], sem).wait()
        pltpu.async_copy(rows_v, buf_hbm.at[i1_v], sem).wait()

    return scatter_kernel(x, idx0, idx1)


def _ralias(e, r, cnt_ref):
    last = jnp.maximum((cnt_ref[e] + R_BLK - 1) // R_BLK, 1) - 1
    return e * N_RBLK + jnp.minimum(r, last)


def _ffn1_body(cnt_ref, buf_ref, w1_ref, h_ref, w1b_ref):
    e = pl.program_id(0)
    r = pl.program_id(1)

    @pl.when(r * R_BLK < cnt_ref[e])
    def _():
        @pl.when(r == 0)
        def _():
            w1b_ref[...] = w1_ref[0].astype(jnp.bfloat16)

        xb = buf_ref[...].astype(jnp.bfloat16)
        h = jnp.dot(xb, w1b_ref[...], preferred_element_type=jnp.float32)
        h_ref[...] = jax.nn.gelu(h).astype(jnp.bfloat16)


def _ffn1(cnt, buf, w1):
    grid_spec = pltpu.PrefetchScalarGridSpec(
        num_scalar_prefetch=1,
        grid=(E, N_RBLK),
        in_specs=[
            pl.BlockSpec((R_BLK, D), lambda e, r, cnt: (_ralias(e, r, cnt), 0)),
            pl.BlockSpec((1, D, F), lambda e, r, cnt: (e, 0, 0)),
        ],
        out_specs=pl.BlockSpec((R_BLK, F), lambda e, r, cnt: (_ralias(e, r, cnt), 0)),
        scratch_shapes=[pltpu.VMEM((D, F), jnp.bfloat16)],
    )
    return pl.pallas_call(
        _ffn1_body,
        grid_spec=grid_spec,
        out_shape=jax.ShapeDtypeStruct((NROWS, F), jnp.bfloat16),
    )(cnt, buf, w1)


def _ffn2_body(cnt_ref, h_ref, w2_ref, y_ref, w2b_ref):
    e = pl.program_id(0)
    r = pl.program_id(1)

    @pl.when(r * R_BLK < cnt_ref[e])
    def _():
        @pl.when(r == 0)
        def _():
            w2b_ref[...] = w2_ref[0].astype(jnp.bfloat16)

        y_ref[...] = jnp.dot(h_ref[...], w2b_ref[...],
                             preferred_element_type=jnp.float32)


def _ffn2(cnt, h, w2):
    grid_spec = pltpu.PrefetchScalarGridSpec(
        num_scalar_prefetch=1,
        grid=(E, N_RBLK),
        in_specs=[
            pl.BlockSpec((R_BLK, F), lambda e, r, cnt: (_ralias(e, r, cnt), 0)),
            pl.BlockSpec((1, F, D), lambda e, r, cnt: (e, 0, 0)),
        ],
        out_specs=pl.BlockSpec((R_BLK, D), lambda e, r, cnt: (_ralias(e, r, cnt), 0)),
        scratch_shapes=[pltpu.VMEM((F, D), jnp.bfloat16)],
    )
    return pl.pallas_call(
        _ffn2_body,
        grid_spec=grid_spec,
        out_shape=jax.ShapeDtypeStruct((NROWS, D), jnp.float32),
    )(cnt, h, w2)


def _ffn(cnt, buf, w1, w2):
    h = _ffn1(cnt, buf, w1)
    return _ffn2(cnt, h, w2)


def _gather2(y, idx0, idx1):
    out_types = (
        jax.ShapeDtypeStruct((T, D), jnp.float32),
        jax.ShapeDtypeStruct((T, D), jnp.float32),
    )

    @functools.partial(
        pl.kernel,
        out_type=out_types,
        mesh=_sc_mesh(),
        scratch_types=[
            pltpu.VMEM((_TPW,), jnp.int32),
            pltpu.VMEM((_TPW, D), jnp.float32),
            pltpu.SemaphoreType.DMA,
        ],
    )
    def gather_kernel(y_hbm, i0_hbm, i1_hbm, a_hbm, b_hbm, i_v, rows_v, sem):
        wid = lax.axis_index("s") * 2 + lax.axis_index("c")
        base = wid * _TPW
        pltpu.sync_copy(i0_hbm.at[pl.ds(base, _TPW)], i_v)
        pltpu.async_copy(y_hbm.at[i_v], rows_v, sem).wait()
        pltpu.sync_copy(rows_v, a_hbm.at[pl.ds(base, _TPW)])
        pltpu.sync_copy(i1_hbm.at[pl.ds(base, _TPW)], i_v)
        pltpu.async_copy(y_hbm.at[i_v], rows_v, sem).wait()
        pltpu.sync_copy(rows_v, b_hbm.at[pl.ds(base, _TPW)])

    return gather_kernel(y, idx0, idx1)


def _combine_body(a_ref, b_ref, g0_ref, g1_ref, o_ref):
    o_ref[...] = g0_ref[...] * a_ref[...] + g1_ref[...] * b_ref[...]


def _combine(a, b, g0, g1):
    n_blk = T // 256
    return pl.pallas_call(
        _combine_body,
        grid=(n_blk,),
        in_specs=[
            pl.BlockSpec((256, D), lambda i: (i, 0)),
            pl.BlockSpec((256, D), lambda i: (i, 0)),
            pl.BlockSpec((256, 1), lambda i: (i, 0)),
            pl.BlockSpec((256, 1), lambda i: (i, 0)),
        ],
        out_specs=pl.BlockSpec((256, D), lambda i: (i, 0)),
        out_shape=jax.ShapeDtypeStruct((T, D), jnp.float32),
    )(a, b, g0, g1)


def kernel(x, w_router, w1, w2):
    wr_pad = jnp.zeros((D, EL), jnp.float32).at[:, :E].set(w_router)
    s0s, s1s, s0g, s1g, g0, g1, cnt = _router(x, wr_pad)
    buf = _dispatch(x, s0s[:, 0], s1s[:, 0])
    y = _ffn(cnt[0], buf, w1, w2)
    a, b = _gather2(y, s0g[:, 0], s1g[:, 0])
    return _combine(a, b, g0, g1)


# speedup vs baseline: 1.7090x; 1.0984x over previous
import functools

import jax
import jax.numpy as jnp
from jax import lax
from jax.experimental import pallas as pl
from jax.experimental.pallas import tpu as pltpu
from jax.experimental.pallas import tpu_sc as plsc

E = 12
K = 2
D = 1024
F = 4096
T = 2048
CAP = (T * K // E) * 2
CAP_PAD = 768
R_BLK = 128
N_RBLK = CAP_PAD // R_BLK
FF_BLK = 512
N_FFBLK = F // FF_BLK
NROWS = E * CAP_PAD
OVERFLOW = NROWS
EL = 128
CHUNK = 128
N_CHUNK = T // CHUNK


def _router_body(x_ref, wr_ref, s0s_ref, s1s_ref, s0g_ref, s1g_ref,
                 g0_ref, g1_ref, cnt_ref):
    x = x_ref[...]
    logits = jnp.dot(x, wr_ref[...], preferred_element_type=jnp.float32)
    col = lax.broadcasted_iota(jnp.int32, (T, EL), 1)
    valid = col < E
    neg = jnp.float32(-1e30)
    logits = jnp.where(valid, logits, neg)
    m = jnp.max(logits, axis=1, keepdims=True)
    ex = jnp.where(valid, jnp.exp(logits - m), 0.0)
    s = jnp.sum(ex, axis=1, keepdims=True)
    p = ex / s
    m0 = jnp.max(p, axis=1, keepdims=True)
    i0 = jnp.min(jnp.where(p == m0, col, EL), axis=1, keepdims=True)
    p2 = jnp.where(col == i0, -1.0, p)
    m1 = jnp.max(p2, axis=1, keepdims=True)
    i1 = jnp.min(jnp.where(p2 == m1, col, EL), axis=1, keepdims=True)
    denom = m0 + m1
    g0 = m0 / denom
    g1 = m1 / denom
    oh0 = (col == i0).astype(jnp.float32)
    oh1 = (col == i1).astype(jnp.float32)
    ohsum = oh0 + oh1
    ri = lax.broadcasted_iota(jnp.int32, (CHUNK, CHUNK), 0)
    ci = lax.broadcasted_iota(jnp.int32, (CHUNK, CHUNK), 1)
    lstrict = (ci < ri).astype(jnp.float32)
    off = jnp.zeros((1, EL), jnp.float32)
    parts = []
    for ci in range(N_CHUNK):
        blk = ohsum[ci * CHUNK:(ci + 1) * CHUNK, :]
        parts.append(jnp.dot(lstrict, blk, preferred_element_type=jnp.float32)
                     + off)
        off = off + jnp.sum(blk, axis=0, keepdims=True)
    cex = jnp.concatenate(parts, axis=0)
    pos0 = jnp.sum(cex * oh0, axis=1, keepdims=True)
    pos1 = jnp.sum(cex * oh1, axis=1, keepdims=True)
    e0 = i0.astype(jnp.float32)
    e1 = i1.astype(jnp.float32)
    keep0 = pos0 < CAP
    keep1 = pos1 < CAP
    slot0 = e0 * CAP_PAD + pos0
    slot1 = e1 * CAP_PAD + pos1
    s0s_ref[...] = jnp.where(keep0, slot0, OVERFLOW).astype(jnp.int32)
    s1s_ref[...] = jnp.where(keep1, slot1, OVERFLOW).astype(jnp.int32)
    s0g_ref[...] = jnp.where(keep0, slot0, 0.0).astype(jnp.int32)
    s1g_ref[...] = jnp.where(keep1, slot1, 0.0).astype(jnp.int32)
    g0_ref[...] = g0 * keep0.astype(jnp.float32)
    g1_ref[...] = g1 * keep1.astype(jnp.float32)
    cnt_ref[...] = jnp.minimum(off, float(CAP)).astype(jnp.int32)


def _router(x, wr_pad):
    out_shapes = (
        jax.ShapeDtypeStruct((T, 1), jnp.int32),
        jax.ShapeDtypeStruct((T, 1), jnp.int32),
        jax.ShapeDtypeStruct((T, 1), jnp.int32),
        jax.ShapeDtypeStruct((T, 1), jnp.int32),
        jax.ShapeDtypeStruct((T, 1), jnp.float32),
        jax.ShapeDtypeStruct((T, 1), jnp.float32),
        jax.ShapeDtypeStruct((1, EL), jnp.int32),
    )
    return pl.pallas_call(_router_body, out_shape=out_shapes)(x, wr_pad)


def _sc_mesh():
    return plsc.VectorSubcoreMesh(core_axis_name="c", subcore_axis_name="s")


_NW = 32
_TPW = T // _NW


def _dispatch(x, idx0, idx1):
    @functools.partial(
        pl.kernel,
        out_type=jax.ShapeDtypeStruct((NROWS + 8, D), jnp.float32),
        mesh=_sc_mesh(),
        scratch_types=[
            pltpu.VMEM((_TPW,), jnp.int32),
            pltpu.VMEM((_TPW,), jnp.int32),
            pltpu.VMEM((_TPW, D), jnp.float32),
            pltpu.SemaphoreType.DMA,
        ],
    )
    def scatter_kernel(x_hbm, i0_hbm, i1_hbm, buf_hbm, i0_v, i1_v, rows_v,
                       sem):
        wid = lax.axis_index("s") * 2 + lax.axis_index("c")
        base = wid * _TPW
        pltpu.sync_copy(i0_hbm.at[pl.ds(base, _TPW)], i0_v)
        pltpu.sync_copy(i1_hbm.at[pl.ds(base, _TPW)], i1_v)
        pltpu.sync_copy(x_hbm.at[pl.ds(base, _TPW)], rows_v)
        pltpu.async_copy(rows_v, buf_hbm.at[i0_v], sem).wait()
        pltpu.async_copy(rows_v, buf_hbm.at[i1_v], sem).wait()

    return scatter_kernel(x, idx0, idx1)


C1 = 4
D_CH = D // C1
C2 = 4
F_CH = F // C2


def _ffn_body(cnt_ref, buf_ref, w1_ref, w2_ref, y_ref, xbuf_ref, hf_ref):
    e = pl.program_id(0)
    s = pl.program_id(1)
    c = pl.program_id(2)

    @pl.when(s == 0)
    def _():
        @pl.when(c == 0)
        def _():
            xbuf_ref[...] = buf_ref[...].astype(jnp.bfloat16)

        w1c = w1_ref[0].astype(jnp.bfloat16)
        for r in range(N_RBLK):
            @pl.when(r * R_BLK < cnt_ref[e])
            def _():
                rs = pl.ds(r * R_BLK, R_BLK)
                part = jnp.dot(xbuf_ref[rs, pl.ds(c * D_CH, D_CH)], w1c,
                               preferred_element_type=jnp.float32)

                @pl.when(c == 0)
                def _():
                    hf_ref[rs, :] = part

                @pl.when(c != 0)
                def _():
                    hf_ref[rs, :] += part

        @pl.when(c == C1 - 1)
        def _():
            for r in range(N_RBLK):
                @pl.when(r * R_BLK < cnt_ref[e])
                def _():
                    rs = pl.ds(r * R_BLK, R_BLK)
                    hf_ref[rs, :] = jax.nn.gelu(hf_ref[rs, :])

    @pl.when(s == 1)
    def _():
        w2c = w2_ref[0].astype(jnp.bfloat16)
        for r in range(N_RBLK):
            @pl.when(r * R_BLK < cnt_ref[e])
            def _():
                rs = pl.ds(r * R_BLK, R_BLK)
                hc = hf_ref[rs, pl.ds(c * F_CH, F_CH)].astype(jnp.bfloat16)
                part = jnp.dot(hc, w2c, preferred_element_type=jnp.float32)

                @pl.when(c == 0)
                def _():
                    y_ref[rs, :] = part

                @pl.when(c != 0)
                def _():
                    y_ref[rs, :] += part


def _ffn(cnt, buf, w1, w2):
    w1c = w1.reshape(E, C1 * D_CH, F).reshape(E * C1, D_CH, F)
    w2c = w2.reshape(E, C2 * F_CH, D).reshape(E * C2, F_CH, D)
    grid_spec = pltpu.PrefetchScalarGridSpec(
        num_scalar_prefetch=1,
        grid=(E, 2, C1),
        in_specs=[
            pl.BlockSpec((CAP_PAD, D),
                         lambda e, s, c, cnt: (jnp.minimum(e + s, E - 1), 0)),
            pl.BlockSpec((1, D_CH, F),
                         lambda e, s, c, cnt:
                         (e * C1 + jnp.where(s == 0, c, C1 - 1), 0, 0)),
            pl.BlockSpec((1, F_CH, D),
                         lambda e, s, c, cnt:
                         (e * C2 + jnp.where(s == 1, c, 0), 0, 0)),
        ],
        out_specs=pl.BlockSpec((CAP_PAD, D), lambda e, s, c, cnt: (e, 0)),
        scratch_shapes=[
            pltpu.VMEM((CAP_PAD, D), jnp.bfloat16),
            pltpu.VMEM((CAP_PAD, F), jnp.float32),
        ],
    )
    return pl.pallas_call(
        _ffn_body,
        grid_spec=grid_spec,
        out_shape=jax.ShapeDtypeStruct((NROWS, D), jnp.float32),
    )(cnt, buf, w1c, w2c)


def _gather2(y, idx0, idx1):
    out_types = (
        jax.ShapeDtypeStruct((T, D), jnp.float32),
        jax.ShapeDtypeStruct((T, D), jnp.float32),
    )

    @functools.partial(
        pl.kernel,
        out_type=out_types,
        mesh=_sc_mesh(),
        scratch_types=[
            pltpu.VMEM((_TPW,), jnp.int32),
            pltpu.VMEM((_TPW, D), jnp.float32),
            pltpu.SemaphoreType.DMA,
        ],
    )
    def gather_kernel(y_hbm, i0_hbm, i1_hbm, a_hbm, b_hbm, i_v, rows_v, sem):
        wid = lax.axis_index("s") * 2 + lax.axis_index("c")
        base = wid * _TPW
        pltpu.sync_copy(i0_hbm.at[pl.ds(base, _TPW)], i_v)
        pltpu.async_copy(y_hbm.at[i_v], rows_v, sem).wait()
        pltpu.sync_copy(rows_v, a_hbm.at[pl.ds(base, _TPW)])
        pltpu.sync_copy(i1_hbm.at[pl.ds(base, _TPW)], i_v)
        pltpu.async_copy(y_hbm.at[i_v], rows_v, sem).wait()
        pltpu.sync_copy(rows_v, b_hbm.at[pl.ds(base, _TPW)])

    return gather_kernel(y, idx0, idx1)


def _combine_body(a_ref, b_ref, g0_ref, g1_ref, o_ref):
    o_ref[...] = g0_ref[...] * a_ref[...] + g1_ref[...] * b_ref[...]


def _combine(a, b, g0, g1):
    n_blk = T // 256
    return pl.pallas_call(
        _combine_body,
        grid=(n_blk,),
        in_specs=[
            pl.BlockSpec((256, D), lambda i: (i, 0)),
            pl.BlockSpec((256, D), lambda i: (i, 0)),
            pl.BlockSpec((256, 1), lambda i: (i, 0)),
            pl.BlockSpec((256, 1), lambda i: (i, 0)),
        ],
        out_specs=pl.BlockSpec((256, D), lambda i: (i, 0)),
        out_shape=jax.ShapeDtypeStruct((T, D), jnp.float32),
    )(a, b, g0, g1)


def kernel(x, w_router, w1, w2):
    wr_pad = jnp.zeros((D, EL), jnp.float32).at[:, :E].set(w_router)
    s0s, s1s, s0g, s1g, g0, g1, cnt = _router(x, wr_pad)
    buf = _dispatch(x, s0s[:, 0], s1s[:, 0])
    y = _ffn(cnt[0], buf, w1, w2)
    a, b = _gather2(y, s0g[:, 0], s1g[:, 0])
    return _combine(a, b, g0, g1)
